# initial kernel scaffold (unmeasured)
import jax
import jax.numpy as jnp
from jax import lax
from jax.experimental import pallas as pl
from jax.experimental.pallas import tpu as pltpu

T = 2048
D = 4096
V_SHARD = 8192
BLK_V = 1024
NV = V_SHARD // BLK_V
NEG_BIG = -1e30


def kernel(x, W, labels):
    x_bf = x.astype(jnp.bfloat16)
    labels2d = labels.reshape(T, 1)

    def body(x_ref, w_ref, lab_ref, out_ref,
             m_ref, s_ref, t_ref, send_buf, recv_buf, send_sem, recv_sem):
        j = pl.program_id(0)
        my_x = lax.axis_index("x")
        my_y = lax.axis_index("y")
        peer = (1 - my_x, my_y)

        @pl.when(j == 0)
        def _():
            barrier = pltpu.get_barrier_semaphore()
            pl.semaphore_signal(
                barrier, inc=1,
                device_id=peer, device_id_type=pl.DeviceIdType.MESH,
            )
            pl.semaphore_wait(barrier, 1)
            m_ref[...] = jnp.full_like(m_ref, NEG_BIG)
            s_ref[...] = jnp.zeros_like(s_ref)
            t_ref[...] = jnp.zeros_like(t_ref)

        w_bf = w_ref[...].astype(jnp.bfloat16)
        logits = jnp.dot(x_ref[...], w_bf, preferred_element_type=jnp.float32)

        m_old = m_ref[...]
        m_new = jnp.maximum(m_old, jnp.max(logits, axis=1, keepdims=True))
        p_sum = jnp.sum(jnp.exp(logits - m_new), axis=1, keepdims=True)
        s_ref[...] = s_ref[...] * jnp.exp(m_old - m_new) + p_sum
        m_ref[...] = m_new

        col0 = my_x * V_SHARD + j * BLK_V
        cols = col0 + lax.broadcasted_iota(jnp.int32, (T, BLK_V), 1)
        hit = cols == lab_ref[...]
        t_ref[...] += jnp.sum(jnp.where(hit, logits, 0.0), axis=1, keepdims=True)

        @pl.when(j == NV - 1)
        def _():
            send_buf[0:1, :] = jnp.transpose(m_ref[...], (1, 0))
            send_buf[1:2, :] = jnp.transpose(s_ref[...], (1, 0))
            send_buf[2:3, :] = jnp.transpose(t_ref[...], (1, 0))
            send_buf[3:8, :] = jnp.zeros((5, T), jnp.float32)
            rdma = pltpu.make_async_remote_copy(
                src_ref=send_buf,
                dst_ref=recv_buf,
                send_sem=send_sem,
                recv_sem=recv_sem,
                device_id=peer,
                device_id_type=pl.DeviceIdType.MESH,
            )
            rdma.start()
            rdma.wait()

            m_l = send_buf[0:1, :]
            s_l = send_buf[1:2, :]
            t_l = send_buf[2:3, :]
            m_p = recv_buf[0:1, :]
            s_p = recv_buf[1:2, :]
            t_p = recv_buf[2:3, :]
            m_g = jnp.maximum(m_l, m_p)
            s_g = s_l * jnp.exp(m_l - m_g) + s_p * jnp.exp(m_p - m_g)
            out_ref[...] = m_g + jnp.log(s_g) - (t_l + t_p)

    out = pl.pallas_call(
        body,
        grid=(NV,),
        out_shape=jax.ShapeDtypeStruct((1, T), jnp.float32),
        in_specs=[
            pl.BlockSpec(memory_space=pltpu.VMEM),
            pl.BlockSpec((D, BLK_V), lambda j: (0, j)),
            pl.BlockSpec(memory_space=pltpu.VMEM),
        ],
        out_specs=pl.BlockSpec(memory_space=pltpu.VMEM),
        scratch_shapes=[
            pltpu.VMEM((T, 1), jnp.float32),
            pltpu.VMEM((T, 1), jnp.float32),
            pltpu.VMEM((T, 1), jnp.float32),
            pltpu.VMEM((8, T), jnp.float32),
            pltpu.VMEM((8, T), jnp.float32),
            pltpu.SemaphoreType.DMA,
            pltpu.SemaphoreType.DMA,
        ],
        compiler_params=pltpu.CompilerParams(
            collective_id=0,
            dimension_semantics=("arbitrary",),
        ),
    )(x_bf, W, labels2d)
    return out.reshape(T)


# baseline (device time: 241994 ns/iter reference)
import jax
import jax.numpy as jnp
from jax import lax
from jax.experimental import pallas as pl
from jax.experimental.pallas import tpu as pltpu

T = 2048
D = 4096
V_SHARD = 8192
BLK_V = 512
NV = V_SHARD // BLK_V
NEG_BIG = -1e30


def kernel(x, W, labels):
    x_bf = x.astype(jnp.bfloat16)
    labels2d = labels.reshape(T, 1)

    def body(x_ref, w_ref, lab_ref, out_ref,
             m_ref, s_ref, t_ref, send_buf, recv_buf, send_sem, recv_sem):
        j = pl.program_id(0)
        my_x = lax.axis_index("x")
        my_y = lax.axis_index("y")
        peer = (1 - my_x, my_y)

        @pl.when(j == 0)
        def _():
            barrier = pltpu.get_barrier_semaphore()
            pl.semaphore_signal(
                barrier, inc=1,
                device_id=peer, device_id_type=pl.DeviceIdType.MESH,
            )
            pl.semaphore_wait(barrier, 1)
            m_ref[...] = jnp.full_like(m_ref, NEG_BIG)
            s_ref[...] = jnp.zeros_like(s_ref)
            t_ref[...] = jnp.zeros_like(t_ref)

        w_bf = w_ref[...].astype(jnp.bfloat16)
        logits = jnp.dot(x_ref[...], w_bf, preferred_element_type=jnp.float32)

        m_old = m_ref[...]
        m_new = jnp.maximum(m_old, jnp.max(logits, axis=1, keepdims=True))
        p_sum = jnp.sum(jnp.exp(logits - m_new), axis=1, keepdims=True)
        s_ref[...] = s_ref[...] * jnp.exp(m_old - m_new) + p_sum
        m_ref[...] = m_new

        col0 = my_x * V_SHARD + j * BLK_V
        cols = col0 + lax.broadcasted_iota(jnp.int32, (T, BLK_V), 1)
        hit = cols == lab_ref[...]
        t_ref[...] += jnp.sum(jnp.where(hit, logits, 0.0), axis=1, keepdims=True)

        @pl.when(j == NV - 1)
        def _():
            send_buf[0:1, :] = jnp.transpose(m_ref[...], (1, 0))
            send_buf[1:2, :] = jnp.transpose(s_ref[...], (1, 0))
            send_buf[2:3, :] = jnp.transpose(t_ref[...], (1, 0))
            send_buf[3:8, :] = jnp.zeros((5, T), jnp.float32)
            rdma = pltpu.make_async_remote_copy(
                src_ref=send_buf,
                dst_ref=recv_buf,
                send_sem=send_sem,
                recv_sem=recv_sem,
                device_id=peer,
                device_id_type=pl.DeviceIdType.MESH,
            )
            rdma.start()
            rdma.wait()

            m_l = send_buf[0:1, :]
            s_l = send_buf[1:2, :]
            t_l = send_buf[2:3, :]
            m_p = recv_buf[0:1, :]
            s_p = recv_buf[1:2, :]
            t_p = recv_buf[2:3, :]
            m_g = jnp.maximum(m_l, m_p)
            s_g = s_l * jnp.exp(m_l - m_g) + s_p * jnp.exp(m_p - m_g)
            out_ref[...] = m_g + jnp.log(s_g) - (t_l + t_p)

    out = pl.pallas_call(
        body,
        grid=(NV,),
        out_shape=jax.ShapeDtypeStruct((1, T), jnp.float32),
        in_specs=[
            pl.BlockSpec(memory_space=pltpu.VMEM),
            pl.BlockSpec((D, BLK_V), lambda j: (0, j)),
            pl.BlockSpec(memory_space=pltpu.VMEM),
        ],
        out_specs=pl.BlockSpec(memory_space=pltpu.VMEM),
        scratch_shapes=[
            pltpu.VMEM((T, 1), jnp.float32),
            pltpu.VMEM((T, 1), jnp.float32),
            pltpu.VMEM((T, 1), jnp.float32),
            pltpu.VMEM((8, T), jnp.float32),
            pltpu.VMEM((8, T), jnp.float32),
            pltpu.SemaphoreType.DMA,
            pltpu.SemaphoreType.DMA,
        ],
        compiler_params=pltpu.CompilerParams(
            collective_id=0,
            dimension_semantics=("arbitrary",),
            vmem_limit_bytes=100 * 1024 * 1024,
        ),
    )(x_bf, W, labels2d)
    return out.reshape(T)


# device time: 187028 ns/iter; 1.2939x vs baseline; 1.2939x over previous
import jax
import jax.numpy as jnp
from jax import lax
from jax.experimental import pallas as pl
from jax.experimental.pallas import tpu as pltpu

T = 2048
D = 4096
V_SHARD = 8192
BLK_V = 512
NV = V_SHARD // BLK_V


def kernel(x, W, labels):
    x_bf = x.astype(jnp.bfloat16)
    labels2d = labels.reshape(T, 1)

    def body(x_ref, w_ref, lab_ref, out_ref,
             s_ref, t_ref, send_buf, recv_buf, send_sem, recv_sem):
        j = pl.program_id(0)
        my_x = lax.axis_index("x")
        my_y = lax.axis_index("y")
        peer = (1 - my_x, my_y)

        @pl.when(j == 0)
        def _():
            barrier = pltpu.get_barrier_semaphore()
            pl.semaphore_signal(
                barrier, inc=1,
                device_id=peer, device_id_type=pl.DeviceIdType.MESH,
            )
            pl.semaphore_wait(barrier, 1)
            s_ref[...] = jnp.zeros_like(s_ref)
            t_ref[...] = jnp.zeros_like(t_ref)

        w_bf = w_ref[...].astype(jnp.bfloat16)
        col0 = my_x * V_SHARD + j * BLK_V

        def lane_fold(a):
            return ((a[:, 0:128] + a[:, 128:256])
                    + (a[:, 256:384] + a[:, 384:512]))

        BLK_T = 512
        for tt in range(T // BLK_T):
            rows = slice(tt * BLK_T, (tt + 1) * BLK_T)
            logits = jnp.dot(
                x_ref[rows, :], w_bf, preferred_element_type=jnp.float32)
            s_ref[rows, :] += lane_fold(jnp.exp(logits))

            li = lax.broadcasted_iota(jnp.int32, (BLK_T, BLK_V), 1)
            hit = li == lab_ref[rows, :] - col0
            t_ref[rows, :] += lane_fold(jnp.where(hit, logits, 0.0))

        @pl.when(j == NV - 1)
        def _():
            s_col = jnp.sum(s_ref[...], axis=1, keepdims=True)
            t_col = jnp.sum(t_ref[...], axis=1, keepdims=True)
            send_buf[0:1, :] = jnp.transpose(s_col, (1, 0))
            send_buf[1:2, :] = jnp.transpose(t_col, (1, 0))
            send_buf[2:8, :] = jnp.zeros((6, T), jnp.float32)
            rdma = pltpu.make_async_remote_copy(
                src_ref=send_buf,
                dst_ref=recv_buf,
                send_sem=send_sem,
                recv_sem=recv_sem,
                device_id=peer,
                device_id_type=pl.DeviceIdType.MESH,
            )
            rdma.start()
            rdma.wait()

            s_g = send_buf[0:1, :] + recv_buf[0:1, :]
            t_g = send_buf[1:2, :] + recv_buf[1:2, :]
            out_ref[...] = jnp.log(s_g) - t_g

    out = pl.pallas_call(
        body,
        grid=(NV,),
        out_shape=jax.ShapeDtypeStruct((1, T), jnp.float32),
        in_specs=[
            pl.BlockSpec(memory_space=pltpu.VMEM),
            pl.BlockSpec((D, BLK_V), lambda j: (0, j)),
            pl.BlockSpec(memory_space=pltpu.VMEM),
        ],
        out_specs=pl.BlockSpec(memory_space=pltpu.VMEM),
        scratch_shapes=[
            pltpu.VMEM((T, 128), jnp.float32),
            pltpu.VMEM((T, 128), jnp.float32),
            pltpu.VMEM((8, T), jnp.float32),
            pltpu.VMEM((8, T), jnp.float32),
            pltpu.SemaphoreType.DMA,
            pltpu.SemaphoreType.DMA,
        ],
        compiler_params=pltpu.CompilerParams(
            collective_id=0,
            dimension_semantics=("arbitrary",),
            vmem_limit_bytes=100 * 1024 * 1024,
        ),
    )(x_bf, W, labels2d)
    return out.reshape(T)
